# baseline (speedup 1.0000x reference)
import functools

import jax
import jax.numpy as jnp
from jax import lax
from jax.experimental import pallas as pl
from jax.experimental.pallas import tpu as pltpu
from jax.experimental.pallas import tpu_sc as plsc


def _rne_bf16_bits(x):
    u = lax.bitcast_convert_type(x, jnp.uint32)
    r = u + jnp.uint32(0x7FFF) + ((u >> 16) & jnp.uint32(1))
    return r & jnp.uint32(0xFFFF0000)


def _node_mlp(x, W1, b1, W2, b2, Wm1ae, Wm1ao, Wm1be, Wm1bo, block_rows,
              interpret=False):
    n, din = x.shape
    hdim = W2.shape[1]
    half = Wm1ae.shape[1]
    grid = n // block_rows

    def body(x_ref, W1_ref, b1_ref, W2_ref, b2_ref,
             Wm1ae_ref, Wm1ao_ref, Wm1be_ref, Wm1bo_ref,
             h_ref, pre1e_ref, pre1o_ref, packed_ref):
        xv = x_ref[...]
        t = jnp.maximum(
            jnp.dot(xv, W1_ref[...], preferred_element_type=jnp.float32)
            + b1_ref[...], 0.0)
        h = (jnp.dot(t, W2_ref[...], preferred_element_type=jnp.float32)
             + b2_ref[...])
        h_ref[...] = h
        pre1e_ref[...] = jnp.dot(h, Wm1ae_ref[...],
                                 preferred_element_type=jnp.float32)
        pre1o_ref[...] = jnp.dot(h, Wm1ao_ref[...],
                                 preferred_element_type=jnp.float32)
        pe = jnp.dot(h, Wm1be_ref[...], preferred_element_type=jnp.float32)
        po = jnp.dot(h, Wm1bo_ref[...], preferred_element_type=jnp.float32)
        packed_u = _rne_bf16_bits(po) | (_rne_bf16_bits(pe) >> 16)
        packed_ref[...] = lax.bitcast_convert_type(packed_u, jnp.float32)

    wspec = pl.BlockSpec((din, hdim), lambda i: (0, 0))
    hspec = pl.BlockSpec((hdim, half), lambda i: (0, 0))
    bspec = pl.BlockSpec((1, hdim), lambda i: (0, 0))
    rspec = pl.BlockSpec((block_rows, half), lambda i: (i, 0))
    return pl.pallas_call(
        body,
        grid=(grid,),
        in_specs=[
            pl.BlockSpec((block_rows, din), lambda i: (i, 0)),
            wspec, bspec,
            pl.BlockSpec((hdim, hdim), lambda i: (0, 0)), bspec,
            hspec, hspec, hspec, hspec,
        ],
        out_specs=[
            pl.BlockSpec((block_rows, hdim), lambda i: (i, 0)),
            rspec, rspec, rspec,
        ],
        out_shape=[
            jax.ShapeDtypeStruct((n, hdim), jnp.float32),
            jax.ShapeDtypeStruct((n, half), jnp.float32),
            jax.ShapeDtypeStruct((n, half), jnp.float32),
            jax.ShapeDtypeStruct((n, half), jnp.float32),
        ],
        interpret=interpret,
    )(x, W1, b1, W2, b2, Wm1ae, Wm1ao, Wm1be, Wm1bo)


def _agg_body(deg, pre1e_ref, pre1o_ref, gp_ref, gc_ref, h_ref,
              Wm2e_ref, Wm2o_ref, bm2_ref,
              cpe_ref, cpo_ref, cce_ref, cco_ref,
              Wap1_ref, bap1_ref, wap2_ref,
              Wac1_ref, bac1_ref, wac2_ref,
              out_ref):
    bn = pre1e_ref.shape[0]
    half = pre1e_ref.shape[1]
    hdim = h_ref.shape[1]
    attn = Wap1_ref.shape[1]
    be = bn * deg
    pre1e = pre1e_ref[...]
    pre1o = pre1o_ref[...]
    hi_mask = jnp.uint32(0xFFFF0000)

    def side(g_ref, ce_ref, co_ref, Wa1_ref, ba1_ref, wa2_ref):
        u = lax.bitcast_convert_type(g_ref[...], jnp.uint32)
        t0 = lax.bitcast_convert_type(u << 16, jnp.float32).reshape(bn, deg, half)
        t1 = lax.bitcast_convert_type(u & hi_mask, jnp.float32).reshape(bn, deg, half)
        e0 = jnp.maximum(t0 + pre1e[:, None, :] + ce_ref[...][None], 0.0)
        e1 = jnp.maximum(t1 + pre1o[:, None, :] + co_ref[...][None], 0.0)
        m = (jnp.dot(e0.reshape(be, half), Wm2e_ref[...],
                     preferred_element_type=jnp.float32)
             + jnp.dot(e1.reshape(be, half), Wm2o_ref[...],
                       preferred_element_type=jnp.float32)
             + bm2_ref[...])
        t = jnp.tanh(jnp.dot(m, Wa1_ref[...],
                             preferred_element_type=jnp.float32) + ba1_ref[...])
        s = jnp.sum(t.reshape(bn, deg, attn) * wa2_ref[...][None], axis=-1)
        s = s - jnp.max(s, axis=-1, keepdims=True)
        p = jnp.exp(s)
        w = p / jnp.sum(p, axis=-1, keepdims=True)
        m3 = m.reshape(bn, deg, hdim)
        return jnp.sum(m3 * w[:, :, None], axis=1)

    up = side(gp_ref, cpe_ref, cpo_ref, Wap1_ref, bap1_ref, wap2_ref)
    down = side(gc_ref, cce_ref, cco_ref, Wac1_ref, bac1_ref, wac2_ref)
    out_ref[...] = jnp.concatenate([h_ref[...], up, down], axis=-1)


def _aggregate(pre1e, pre1o, gp, gc, h, Wm2e, Wm2o, bm2,
               cpe, cpo, cce, cco,
               Wap1, bap1, wap2, Wac1, bac1, wac2, block_nodes, deg,
               interpret=False):
    n, half = pre1e.shape
    hdim = h.shape[1]
    attn = Wap1.shape[1]
    grid = n // block_nodes
    be = block_nodes * deg
    const_spec = pl.BlockSpec((1, half), lambda i: (0, 0))
    attn_spec = pl.BlockSpec((1, attn), lambda i: (0, 0))
    return pl.pallas_call(
        functools.partial(_agg_body, deg),
        grid=(grid,),
        in_specs=[
            pl.BlockSpec((block_nodes, half), lambda i: (i, 0)),
            pl.BlockSpec((block_nodes, half), lambda i: (i, 0)),
            pl.BlockSpec((be, half), lambda i: (i, 0)),
            pl.BlockSpec((be, half), lambda i: (i, 0)),
            pl.BlockSpec((block_nodes, hdim), lambda i: (i, 0)),
            pl.BlockSpec((half, hdim), lambda i: (0, 0)),
            pl.BlockSpec((half, hdim), lambda i: (0, 0)),
            pl.BlockSpec((1, hdim), lambda i: (0, 0)),
            const_spec, const_spec, const_spec, const_spec,
            pl.BlockSpec((hdim, attn), lambda i: (0, 0)),
            attn_spec, attn_spec,
            pl.BlockSpec((hdim, attn), lambda i: (0, 0)),
            attn_spec, attn_spec,
        ],
        out_specs=pl.BlockSpec((block_nodes, 3 * hdim), lambda i: (i, 0)),
        out_shape=jax.ShapeDtypeStruct((n, 3 * hdim), jnp.float32),
        interpret=interpret,
    )(pre1e, pre1o, gp, gc, h, Wm2e, Wm2o, bm2, cpe, cpo, cce, cco,
      Wap1, bap1, wap2, Wac1, bac1, wac2)


_CHUNK = 128


_SLOW_CORE = 1
_SLOW_FRAC = 1.0


def _sc_gather(table, idx_flat):
    tchunks, ch = idx_flat.shape
    rows = tchunks * ch
    d = table.shape[1]
    per_tile = tchunks // 16
    a = int(per_tile * _SLOW_FRAC * 0.5) & ~1
    b = per_tile - a
    assert a % 2 == 0 and b % 2 == 0 and b >= a
    mesh = plsc.VectorSubcoreMesh(core_axis_name="c", subcore_axis_name="s")

    @functools.partial(
        pl.kernel,
        mesh=mesh,
        out_type=jax.ShapeDtypeStruct((rows, d), jnp.float32),
        scratch_types=[
            pltpu.VMEM((b, ch), jnp.int32),
            pltpu.VMEM((ch, d), jnp.float32),
            pltpu.VMEM((ch, d), jnp.float32),
            pltpu.SemaphoreType.DMA,
            pltpu.SemaphoreType.DMA,
        ],
    )
    def gather_k(table_hbm, idx_hbm, out_hbm, idx_v, rows_a, rows_b, sem_a, sem_b):
        c = lax.axis_index("c")
        s = lax.axis_index("s")
        is_slow = c == _SLOW_CORE
        myn = jnp.where(is_slow, a, b)
        start = jnp.where(is_slow, s * a, 16 * a + s * b)
        pltpu.sync_copy(idx_hbm.at[pl.ds(start, b)], idx_v)
        base = start * ch

        pltpu.make_async_copy(table_hbm.at[idx_v.at[0]], rows_a, sem_a).start()

        def body(i, _):
            j = 2 * i
            pltpu.make_async_copy(table_hbm.at[idx_v.at[j + 1]], rows_b, sem_b).start()
            pltpu.make_async_copy(table_hbm.at[idx_v.at[j]], rows_a, sem_a).wait()
            pltpu.sync_copy(rows_a, out_hbm.at[pl.ds(base + j * ch, ch)])

            @pl.when(j + 2 < myn)
            def _():
                pltpu.make_async_copy(
                    table_hbm.at[idx_v.at[j + 2]], rows_a, sem_a).start()

            pltpu.make_async_copy(table_hbm.at[idx_v.at[j + 1]], rows_b, sem_b).wait()
            pltpu.sync_copy(rows_b, out_hbm.at[pl.ds(base + (j + 1) * ch, ch)])
            return 0

        lax.fori_loop(0, myn // 2, body, 0)

    return gather_k(table, idx_flat)


_SLICES = 5


def kernel(x_nodes, parents_list, children_list,
           W1, b1, W2, b2,
           Wm1, bm1, Wm2, bm2,
           Wap1, bap1, Wap2, bap2,
           Wac1, bac1, Wac2, bac2):
    n, din = x_nodes.shape
    deg = parents_list.shape[1]
    hdim = W2.shape[1]
    msg = Wm2.shape[0]
    attn = Wap1.shape[1]
    half = msg // 2

    Wm1a = Wm1[:hdim]
    Wm1b = Wm1[hdim:2 * hdim]
    h, pre1e, pre1o, packed = _node_mlp(
        x_nodes, W1, b1.reshape(1, hdim), W2, b2.reshape(1, hdim),
        Wm1a[:, 0::2], Wm1a[:, 1::2], Wm1b[:, 0::2], Wm1b[:, 1::2],
        block_rows=1000)

    cp = (bm1 + Wm1[2 * hdim])
    cc = bm1
    agg_consts = (Wm2[0::2], Wm2[1::2], bm2.reshape(1, hdim),
                  cp[0::2].reshape(1, half), cp[1::2].reshape(1, half),
                  cc[0::2].reshape(1, half), cc[1::2].reshape(1, half),
                  Wap1, bap1.reshape(1, attn), Wap2.reshape(1, attn),
                  Wac1, bac1.reshape(1, attn), Wac2.reshape(1, attn))

    ns = n // _SLICES
    outs = []
    for si in range(_SLICES):
        lo = si * ns
        idx_s = jnp.concatenate(
            [lax.dynamic_slice_in_dim(parents_list, lo, ns).reshape(-1),
             lax.dynamic_slice_in_dim(children_list, lo, ns).reshape(-1)]
        ).astype(jnp.int32)
        ne_s = ns * deg
        total = 2 * ne_s
        tchunks = -(-total // _CHUNK)
        tchunks += -tchunks % 64
        idx_s = jnp.pad(idx_s, (0, tchunks * _CHUNK - total))
        gathered = _sc_gather(packed, idx_s.reshape(tchunks, _CHUNK))
        outs.append(_aggregate(
            pre1e[lo:lo + ns], pre1o[lo:lo + ns],
            gathered[:ne_s], gathered[ne_s:2 * ne_s], h[lo:lo + ns],
            *agg_consts, block_nodes=200, deg=deg))
    return jnp.concatenate(outs, axis=0)


# speedup vs baseline: 1.1134x; 1.1134x over previous
import functools

import jax
import jax.numpy as jnp
from jax import lax
from jax.experimental import pallas as pl
from jax.experimental.pallas import tpu as pltpu
from jax.experimental.pallas import tpu_sc as plsc


def _rne_bf16_bits(x):
    u = lax.bitcast_convert_type(x, jnp.uint32)
    r = u + jnp.uint32(0x7FFF) + ((u >> 16) & jnp.uint32(1))
    return r & jnp.uint32(0xFFFF0000)


def _node_mlp(x, W1, b1, W2, b2, Wm1ae, Wm1ao, Wm1be, Wm1bo, block_rows,
              interpret=False):
    n, din = x.shape
    hdim = W2.shape[1]
    half = Wm1ae.shape[1]
    grid = n // block_rows

    def body(x_ref, W1_ref, b1_ref, W2_ref, b2_ref,
             Wm1ae_ref, Wm1ao_ref, Wm1be_ref, Wm1bo_ref,
             h_ref, pre1e_ref, pre1o_ref, packed_ref):
        xv = x_ref[...]
        t = jnp.maximum(
            jnp.dot(xv, W1_ref[...], preferred_element_type=jnp.float32)
            + b1_ref[...], 0.0)
        h = (jnp.dot(t, W2_ref[...], preferred_element_type=jnp.float32)
             + b2_ref[...])
        h_ref[...] = h
        pre1e_ref[...] = jnp.dot(h, Wm1ae_ref[...],
                                 preferred_element_type=jnp.float32)
        pre1o_ref[...] = jnp.dot(h, Wm1ao_ref[...],
                                 preferred_element_type=jnp.float32)
        pe = jnp.dot(h, Wm1be_ref[...], preferred_element_type=jnp.float32)
        po = jnp.dot(h, Wm1bo_ref[...], preferred_element_type=jnp.float32)
        packed_u = _rne_bf16_bits(po) | (_rne_bf16_bits(pe) >> 16)
        packed_ref[...] = lax.bitcast_convert_type(packed_u, jnp.float32)

    wspec = pl.BlockSpec((din, hdim), lambda i: (0, 0))
    hspec = pl.BlockSpec((hdim, half), lambda i: (0, 0))
    bspec = pl.BlockSpec((1, hdim), lambda i: (0, 0))
    rspec = pl.BlockSpec((block_rows, half), lambda i: (i, 0))
    return pl.pallas_call(
        body,
        grid=(grid,),
        in_specs=[
            pl.BlockSpec((block_rows, din), lambda i: (i, 0)),
            wspec, bspec,
            pl.BlockSpec((hdim, hdim), lambda i: (0, 0)), bspec,
            hspec, hspec, hspec, hspec,
        ],
        out_specs=[
            pl.BlockSpec((block_rows, hdim), lambda i: (i, 0)),
            rspec, rspec, rspec,
        ],
        out_shape=[
            jax.ShapeDtypeStruct((n, hdim), jnp.float32),
            jax.ShapeDtypeStruct((n, half), jnp.float32),
            jax.ShapeDtypeStruct((n, half), jnp.float32),
            jax.ShapeDtypeStruct((n, half), jnp.float32),
        ],
        interpret=interpret,
    )(x, W1, b1, W2, b2, Wm1ae, Wm1ao, Wm1be, Wm1bo)


def _agg_body(deg, pre1e_ref, pre1o_ref, gp_ref, gc_ref, h_ref,
              Wm2e_ref, Wm2o_ref, bm2_ref,
              cpe_ref, cpo_ref, cce_ref, cco_ref,
              Wap1_ref, bap1_ref, wap2_ref,
              Wac1_ref, bac1_ref, wac2_ref,
              out_ref):
    bn = pre1e_ref.shape[0]
    half = pre1e_ref.shape[1]
    hdim = h_ref.shape[1]
    attn = Wap1_ref.shape[1]
    be = bn * deg
    pre1e = pre1e_ref[...]
    pre1o = pre1o_ref[...]
    hi_mask = jnp.uint32(0xFFFF0000)

    def side(g_ref, ce_ref, co_ref, Wa1_ref, ba1_ref, wa2_ref):
        u = lax.bitcast_convert_type(g_ref[...], jnp.uint32)
        t0 = lax.bitcast_convert_type(u << 16, jnp.float32).reshape(bn, deg, half)
        t1 = lax.bitcast_convert_type(u & hi_mask, jnp.float32).reshape(bn, deg, half)
        e0 = jnp.maximum(t0 + pre1e[:, None, :] + ce_ref[...][None], 0.0)
        e1 = jnp.maximum(t1 + pre1o[:, None, :] + co_ref[...][None], 0.0)
        m = (jnp.dot(e0.reshape(be, half), Wm2e_ref[...],
                     preferred_element_type=jnp.float32)
             + jnp.dot(e1.reshape(be, half), Wm2o_ref[...],
                       preferred_element_type=jnp.float32)
             + bm2_ref[...])
        t = jnp.tanh(jnp.dot(m, Wa1_ref[...],
                             preferred_element_type=jnp.float32) + ba1_ref[...])
        s = jnp.sum(t.reshape(bn, deg, attn) * wa2_ref[...][None], axis=-1)
        s = s - jnp.max(s, axis=-1, keepdims=True)
        p = jnp.exp(s)
        w = p / jnp.sum(p, axis=-1, keepdims=True)
        m3 = m.reshape(bn, deg, hdim)
        return jnp.sum(m3 * w[:, :, None], axis=1)

    up = side(gp_ref, cpe_ref, cpo_ref, Wap1_ref, bap1_ref, wap2_ref)
    down = side(gc_ref, cce_ref, cco_ref, Wac1_ref, bac1_ref, wac2_ref)
    out_ref[...] = jnp.concatenate([h_ref[...], up, down], axis=-1)


def _aggregate(pre1e, pre1o, gp, gc, h, Wm2e, Wm2o, bm2,
               cpe, cpo, cce, cco,
               Wap1, bap1, wap2, Wac1, bac1, wac2, block_nodes, deg,
               interpret=False):
    n, half = pre1e.shape
    hdim = h.shape[1]
    attn = Wap1.shape[1]
    grid = n // block_nodes
    be = block_nodes * deg
    const_spec = pl.BlockSpec((1, half), lambda i: (0, 0))
    attn_spec = pl.BlockSpec((1, attn), lambda i: (0, 0))
    return pl.pallas_call(
        functools.partial(_agg_body, deg),
        grid=(grid,),
        in_specs=[
            pl.BlockSpec((block_nodes, half), lambda i: (i, 0)),
            pl.BlockSpec((block_nodes, half), lambda i: (i, 0)),
            pl.BlockSpec((be, half), lambda i: (i, 0)),
            pl.BlockSpec((be, half), lambda i: (i, 0)),
            pl.BlockSpec((block_nodes, hdim), lambda i: (i, 0)),
            pl.BlockSpec((half, hdim), lambda i: (0, 0)),
            pl.BlockSpec((half, hdim), lambda i: (0, 0)),
            pl.BlockSpec((1, hdim), lambda i: (0, 0)),
            const_spec, const_spec, const_spec, const_spec,
            pl.BlockSpec((hdim, attn), lambda i: (0, 0)),
            attn_spec, attn_spec,
            pl.BlockSpec((hdim, attn), lambda i: (0, 0)),
            attn_spec, attn_spec,
        ],
        out_specs=pl.BlockSpec((block_nodes, 3 * hdim), lambda i: (i, 0)),
        out_shape=jax.ShapeDtypeStruct((n, 3 * hdim), jnp.float32),
        interpret=interpret,
    )(pre1e, pre1o, gp, gc, h, Wm2e, Wm2o, bm2, cpe, cpo, cce, cco,
      Wap1, bap1, wap2, Wac1, bac1, wac2)


_CHUNK = 128


_SLOW_CORE = 1
_SLOW_FRAC = 1.0


def _sc_gather(table, idx_flat):
    tchunks, ch = idx_flat.shape
    rows = tchunks * ch
    d = table.shape[1]
    per_tile = tchunks // 16
    a = int(per_tile * _SLOW_FRAC * 0.5) & ~1
    b = per_tile - a
    assert a % 2 == 0 and b % 2 == 0 and b >= a
    mesh = plsc.VectorSubcoreMesh(core_axis_name="c", subcore_axis_name="s")

    @functools.partial(
        pl.kernel,
        mesh=mesh,
        out_type=jax.ShapeDtypeStruct((rows, d), jnp.float32),
        scratch_types=[
            pltpu.VMEM((b, ch), jnp.int32),
            pltpu.VMEM((ch, d), jnp.float32),
            pltpu.VMEM((ch, d), jnp.float32),
            pltpu.SemaphoreType.DMA,
            pltpu.SemaphoreType.DMA,
        ],
    )
    def gather_k(table_hbm, idx_hbm, out_hbm, idx_v, rows_a, rows_b, sem_a, sem_b):
        c = lax.axis_index("c")
        s = lax.axis_index("s")
        is_slow = c == _SLOW_CORE
        myn = jnp.where(is_slow, a, b)
        start = jnp.where(is_slow, s * a, 16 * a + s * b)
        pltpu.sync_copy(idx_hbm.at[pl.ds(start, b)], idx_v)
        base = start * ch

        pltpu.make_async_copy(table_hbm.at[idx_v.at[0]], rows_a, sem_a).start()

        def body(i, _):
            j = 2 * i
            pltpu.make_async_copy(table_hbm.at[idx_v.at[j + 1]], rows_b, sem_b).start()
            pltpu.make_async_copy(table_hbm.at[idx_v.at[j]], rows_a, sem_a).wait()
            pltpu.sync_copy(rows_a, out_hbm.at[pl.ds(base + j * ch, ch)])

            @pl.when(j + 2 < myn)
            def _():
                pltpu.make_async_copy(
                    table_hbm.at[idx_v.at[j + 2]], rows_a, sem_a).start()

            pltpu.make_async_copy(table_hbm.at[idx_v.at[j + 1]], rows_b, sem_b).wait()
            pltpu.sync_copy(rows_b, out_hbm.at[pl.ds(base + (j + 1) * ch, ch)])
            return 0

        lax.fori_loop(0, myn // 2, body, 0)

    return gather_k(table, idx_flat)


_SLICES = 2


def kernel(x_nodes, parents_list, children_list,
           W1, b1, W2, b2,
           Wm1, bm1, Wm2, bm2,
           Wap1, bap1, Wap2, bap2,
           Wac1, bac1, Wac2, bac2):
    n, din = x_nodes.shape
    deg = parents_list.shape[1]
    hdim = W2.shape[1]
    msg = Wm2.shape[0]
    attn = Wap1.shape[1]
    half = msg // 2

    Wm1a = Wm1[:hdim]
    Wm1b = Wm1[hdim:2 * hdim]
    h, pre1e, pre1o, packed = _node_mlp(
        x_nodes, W1, b1.reshape(1, hdim), W2, b2.reshape(1, hdim),
        Wm1a[:, 0::2], Wm1a[:, 1::2], Wm1b[:, 0::2], Wm1b[:, 1::2],
        block_rows=1000)

    cp = (bm1 + Wm1[2 * hdim])
    cc = bm1
    agg_consts = (Wm2[0::2], Wm2[1::2], bm2.reshape(1, hdim),
                  cp[0::2].reshape(1, half), cp[1::2].reshape(1, half),
                  cc[0::2].reshape(1, half), cc[1::2].reshape(1, half),
                  Wap1, bap1.reshape(1, attn), Wap2.reshape(1, attn),
                  Wac1, bac1.reshape(1, attn), Wac2.reshape(1, attn))

    ns = n // _SLICES
    outs = []
    for si in range(_SLICES):
        lo = si * ns
        idx_s = jnp.concatenate(
            [lax.dynamic_slice_in_dim(parents_list, lo, ns).reshape(-1),
             lax.dynamic_slice_in_dim(children_list, lo, ns).reshape(-1)]
        ).astype(jnp.int32)
        ne_s = ns * deg
        total = 2 * ne_s
        tchunks = -(-total // _CHUNK)
        tchunks += -tchunks % 64
        idx_s = jnp.pad(idx_s, (0, tchunks * _CHUNK - total))
        gathered = _sc_gather(packed, idx_s.reshape(tchunks, _CHUNK))
        outs.append(_aggregate(
            pre1e[lo:lo + ns], pre1o[lo:lo + ns],
            gathered[:ne_s], gathered[ne_s:2 * ne_s], h[lo:lo + ns],
            *agg_consts, block_nodes=200, deg=deg))
    return jnp.concatenate(outs, axis=0)
